# Initial kernel scaffold; baseline (speedup 1.0000x reference)
import jax
import jax.numpy as jnp
from jax.experimental import pallas as pl


def kernel(nodes_feature, e, mask, t, edge_index, params):
    raise NotImplementedError("write your pallas kernel here")


# speedup vs baseline: 3.7561x; 3.7561x over previous
import functools

import jax
import jax.numpy as jnp
import numpy as np
from jax import lax
from jax.experimental import pallas as pl
from jax.experimental.pallas import tpu as pltpu
from jax.experimental.pallas import tpu_sc as plsc

_N = 10000
_E = 160000
_H = 128
_TD = _H // 2
_L = 12

_TE = 2000
_TN = 2000

_NC = 2
_NS = 16
_NW = _NC * _NS

_PW = _E // _NW
_PF = _PW // 128
_PR = _PW - _PF * 128
_ZR = 632
_ZL = _N - 15 * _ZR

_f32 = jnp.float32


@functools.lru_cache(maxsize=None)
def _sc_kernels():
    mesh = plsc.VectorSubcoreMesh(core_axis_name="c", subcore_axis_name="s")

    @functools.partial(
        pl.kernel,
        out_type=(jax.ShapeDtypeStruct((_E, _H), _f32),
                  jax.ShapeDtypeStruct((_E, _H), _f32)),
        mesh=mesh,
        scratch_types=[
            pltpu.VMEM((128,), jnp.int32),
            pltpu.VMEM((128, _H), _f32),
            pltpu.VMEM((_PR,), jnp.int32),
            pltpu.VMEM((_PR, _H), _f32),
            pltpu.SemaphoreType.DMA,
        ],
    )
    def sc_gather(x_hbm, dst_hbm, src_hbm, xd_hbm, xs_hbm,
                  idx_v, rows_v, idx_r, rows_r, sem):
        wid = lax.axis_index("s") * _NC + lax.axis_index("c")
        base = wid * _PW

        def run(tab_hbm, out_hbm):
            @pl.loop(0, _PF)
            def _(i):
                off = base + i * 128
                pltpu.sync_copy(tab_hbm.at[pl.ds(off, 128)], idx_v)
                pltpu.async_copy(x_hbm.at[idx_v], rows_v, sem).wait()
                pltpu.sync_copy(rows_v, out_hbm.at[pl.ds(off, 128)])

            off = base + _PF * 128
            pltpu.sync_copy(tab_hbm.at[pl.ds(off, _PR)], idx_r)
            pltpu.async_copy(x_hbm.at[idx_r], rows_r, sem).wait()
            pltpu.sync_copy(rows_r, out_hbm.at[pl.ds(off, _PR)])

        run(dst_hbm, xd_hbm)
        run(src_hbm, xs_hbm)

    @functools.partial(
        pl.kernel,
        out_type=jax.ShapeDtypeStruct((_NC, _N, _H), _f32),
        mesh=mesh,
        scratch_types=[
            pltpu.VMEM((128,), jnp.int32),
            pltpu.VMEM((128, _H), _f32),
            pltpu.VMEM((_PR,), jnp.int32),
            pltpu.VMEM((_PR, _H), _f32),
            pltpu.VMEM_SHARED((_N, _H), _f32),
            pltpu.SemaphoreType.DMA,
        ],
    )
    def sc_scatter(m_hbm, src_hbm, zero_hbm, out_hbm,
                   idx_v, rows_v, idx_r, rows_r, acc, sem):
        cid = lax.axis_index("c")
        sid = lax.axis_index("s")
        z0 = sid * _ZR

        @pl.when(sid < 15)
        def _():
            pltpu.sync_copy(zero_hbm.at[pl.ds(z0, _ZR)], acc.at[pl.ds(z0, _ZR)])

        @pl.when(sid == 15)
        def _():
            pltpu.sync_copy(zero_hbm.at[pl.ds(z0, _ZL)], acc.at[pl.ds(z0, _ZL)])

        plsc.subcore_barrier()

        base = cid * (_E // _NC) + sid * _PW

        @pl.loop(0, _PF)
        def _(i):
            off = base + i * 128
            pltpu.sync_copy(src_hbm.at[pl.ds(off, 128)], idx_v)
            pltpu.sync_copy(m_hbm.at[pl.ds(off, 128)], rows_v)
            pltpu.sync_copy(rows_v, acc.at[idx_v], add=True)

        off = base + _PF * 128
        pltpu.sync_copy(src_hbm.at[pl.ds(off, _PR)], idx_r)
        pltpu.sync_copy(m_hbm.at[pl.ds(off, _PR)], rows_r)
        pltpu.sync_copy(rows_r, acc.at[idx_r], add=True)

        plsc.subcore_barrier()

        @pl.when(sid < 15)
        def _():
            pltpu.sync_copy(acc.at[pl.ds(z0, _ZR)],
                            out_hbm.at[cid, pl.ds(z0, _ZR)])

        @pl.when(sid == 15)
        def _():
            pltpu.sync_copy(acc.at[pl.ds(z0, _ZL)],
                            out_hbm.at[cid, pl.ds(z0, _ZL)])

    return sc_gather, sc_scatter


def _ln(x, g, b):
    mu = jnp.mean(x, axis=-1, keepdims=True)
    xc = x - mu
    var = jnp.mean(xc * xc, axis=-1, keepdims=True)
    return xc * lax.rsqrt(var + 1e-5) * g + b


def _mm(x, w):
    return lax.dot_general(x, w, (((1,), (1,)), ((), ())))


def _edge_body(ee_ref, xd_ref, xs_ref, cw, cb, aw, ab, bw, bb, vw, vb,
               lneg, lneb, tt, plog, plob, plow, plob2, eeo_ref, m_ref):
    ee = ee_ref[...]
    xd = xd_ref[...]
    xs = xs_ref[...]
    ce = _mm(ee, cw[...]) + cb[...]
    ah = _mm(xd, aw[...]) + ab[...]
    bh = _mm(xs, bw[...]) + bb[...]
    vh = _mm(xd, vw[...]) + vb[...]
    e_new = ah + bh + ce
    m_ref[...] = vh * jax.nn.sigmoid(e_new)
    t1 = jax.nn.relu(_ln(e_new, lneg[...], lneb[...]))
    t2 = t1 + tt[...]
    eo = _ln(t2, plog[...], plob[...])
    eo = eo * jax.nn.sigmoid(eo)
    eo = _mm(eo, plow[...]) + plob2[...]
    eeo_ref[...] = ee + eo


def _node_body(x_ref, agg_ref, uw, ub, lnhg, lnhb, xo_ref):
    x = x_ref[...]
    uh = _mm(x, uw[...]) + ub[...]
    s = uh + agg_ref[0] + agg_ref[1]
    xo_ref[...] = x + jax.nn.relu(_ln(s, lnhg[...], lnhb[...]))


def _x0_body(coords_ref, freq_ref, msk_ref, w, b, xo_ref):
    yc = coords_ref[:, 0:1]
    xc = coords_ref[:, 1:2]
    sel = msk_ref[0:1, :]
    par = msk_ref[1:2, :]
    c = yc * sel + xc * (1.0 - sel)
    arg = c * freq_ref[...]
    feat = jnp.where(par > 0.0, jnp.sin(arg), jnp.cos(arg))
    xo_ref[...] = _mm(feat, w[...]) + b[...]


def _ee0_body(e_ref, mask_ref, freq_ref, msk_ref, w, b, me, eeo_ref):
    arg = e_ref[...] * freq_ref[...]
    par = msk_ref[1:2, :]
    feat = jnp.where(par > 0.0, jnp.sin(arg), jnp.cos(arg))
    emb = _mm(feat, w[...]) + b[...]
    mk = mask_ref[...]
    me0 = me[0:1, :]
    me1 = me[1:2, :]
    eeo_ref[...] = emb + me0 + mk * (me1 - me0)


def _tt_body(t_ref, tfreq, w1, b1, w2, b2, tlw, tlb, out_ref):
    targ = t_ref[...] * tfreq[...]
    feat = jnp.concatenate([jnp.cos(targ), jnp.sin(targ)], axis=1)
    h1 = jax.nn.relu(_mm(feat, w1[...]) + b1[...])
    h2 = _mm(h1, w2[...]) + b2[...]
    rt = jax.nn.relu(h2)
    out_ref[...] = jnp.dot(rt, tlw[...]) + tlb[...]


def _stats_body(ee_ref, out_ref):
    @pl.when(pl.program_id(0) == 0)
    def _():
        out_ref[...] = jnp.zeros_like(out_ref)

    ee = ee_ref[...]
    out_ref[0:1, :] += jnp.sum(ee, axis=0, keepdims=True)
    out_ref[1:2, :] += jnp.sum(ee * ee, axis=0, keepdims=True)


def _proj_body(ee_ref, st_ref, gmat, gng, gnb, ow, ob, out_ref):
    mu = jnp.dot(st_ref[0:1, :], gmat[...])
    ms = jnp.dot(st_ref[1:2, :], gmat[...])
    var = ms - mu * mu
    inv = lax.rsqrt(var + 1e-5)
    yv = jax.nn.relu((ee_ref[...] - mu) * inv * gng[...] + gnb[...])
    out_ref[...] = _mm(yv, ow[...]) + ob[...]


def _full(i):
    return (0, 0)


def _rows(i):
    return (i, 0)


def _spec(shape, imap):
    return pl.BlockSpec(shape, imap)


_EDGE_GRID = _E // _TE
_NODE_GRID = _N // _TN

_edge_call = pl.pallas_call(
    _edge_body,
    grid=(_EDGE_GRID,),
    in_specs=[
        _spec((_TE, _H), _rows), _spec((_TE, _H), _rows), _spec((_TE, _H), _rows),
        _spec((_H, _H), _full), _spec((1, _H), _full),
        _spec((_H, _H), _full), _spec((1, _H), _full),
        _spec((_H, _H), _full), _spec((1, _H), _full),
        _spec((_H, _H), _full), _spec((1, _H), _full),
        _spec((1, _H), _full), _spec((1, _H), _full),
        _spec((1, _H), _full),
        _spec((1, _H), _full), _spec((1, _H), _full),
        _spec((_H, _H), _full), _spec((1, _H), _full),
    ],
    out_specs=[_spec((_TE, _H), _rows), _spec((_TE, _H), _rows)],
    out_shape=[jax.ShapeDtypeStruct((_E, _H), _f32),
               jax.ShapeDtypeStruct((_E, _H), _f32)],
)

_node_call = pl.pallas_call(
    _node_body,
    grid=(_NODE_GRID,),
    in_specs=[
        _spec((_TN, _H), _rows),
        pl.BlockSpec((_NC, _TN, _H), lambda i: (0, i, 0)),
        _spec((_H, _H), _full), _spec((1, _H), _full),
        _spec((1, _H), _full), _spec((1, _H), _full),
    ],
    out_specs=_spec((_TN, _H), _rows),
    out_shape=jax.ShapeDtypeStruct((_N, _H), _f32),
)

_x0_call = pl.pallas_call(
    _x0_body,
    grid=(_NODE_GRID,),
    in_specs=[
        _spec((_TN, 2), _rows),
        _spec((1, _H), _full), _spec((2, _H), _full),
        _spec((_H, _H), _full), _spec((1, _H), _full),
    ],
    out_specs=_spec((_TN, _H), _rows),
    out_shape=jax.ShapeDtypeStruct((_N, _H), _f32),
)

_ee0_call = pl.pallas_call(
    _ee0_body,
    grid=(_EDGE_GRID,),
    in_specs=[
        _spec((_TE, 1), _rows), _spec((_TE, 1), _rows),
        _spec((1, _H), _full), _spec((2, _H), _full),
        _spec((_H, _H), _full), _spec((1, _H), _full),
        _spec((2, _H), _full),
    ],
    out_specs=_spec((_TE, _H), _rows),
    out_shape=jax.ShapeDtypeStruct((_E, _H), _f32),
)

_tt_call = pl.pallas_call(
    _tt_body,
    grid=(1,),
    in_specs=[
        _spec((1, 1), _full), _spec((1, _TD), _full),
        _spec((_TD, _H), _full), _spec((1, _TD), _full),
        _spec((_TD, _TD), _full), _spec((1, _TD), _full),
        _spec((_TD, _L * _H), _full), _spec((1, _L * _H), _full),
    ],
    out_specs=_spec((1, _L * _H), _full),
    out_shape=jax.ShapeDtypeStruct((1, _L * _H), _f32),
)

_stats_call = pl.pallas_call(
    _stats_body,
    grid=(_EDGE_GRID,),
    in_specs=[_spec((_TE, _H), _rows)],
    out_specs=_spec((8, _H), _full),
    out_shape=jax.ShapeDtypeStruct((8, _H), _f32),
)

_proj_call = pl.pallas_call(
    _proj_body,
    grid=(_EDGE_GRID,),
    in_specs=[
        _spec((_TE, _H), _rows), _spec((8, _H), _full),
        _spec((_H, _H), _full),
        _spec((1, _H), _full), _spec((1, _H), _full),
        _spec((8, _H), _full), _spec((1, 8), _full),
    ],
    out_specs=_spec((_TE, 8), _rows),
    out_shape=jax.ShapeDtypeStruct((_E, 8), _f32),
)


def _np_consts():
    npf = _H // 2
    dim64 = 10000.0 ** (2.0 * np.floor(np.arange(npf) / 2.0) / npf)
    freq_pos = np.tile(2.0 * np.pi / dim64, 2).astype(np.float32)[None, :]
    dim128 = 10000.0 ** (2.0 * np.floor(np.arange(_H) / 2.0) / _H)
    freq_e = (1.0 / dim128).astype(np.float32)[None, :]
    j = np.arange(_H)
    msk = np.stack([(j < npf).astype(np.float32),
                    (j % 2 == 0).astype(np.float32)]).astype(np.float32)
    tfreq = np.exp(-np.log(10000.0) * np.arange(_TD) / _TD).astype(np.float32)[None, :]
    gid = j // 4
    gmat = ((gid[:, None] == gid[None, :]).astype(np.float32) / (4.0 * _E))
    return freq_pos, freq_e, msk, tfreq, gmat


def kernel(nodes_feature, e, mask, t, edge_index, params):
    p = params
    freq_pos, freq_e, msk, tfreq, gmat = _np_consts()

    src = edge_index[0]
    dst = edge_index[1]

    def b2(v):
        return v.reshape(1, -1)

    x = _x0_call(nodes_feature, freq_pos, msk,
                 p['node_embed_W'], b2(p['node_embed_b']))
    ee = _ee0_call(e.reshape(_E, 1), mask.astype(_f32).reshape(_E, 1),
                   freq_e, msk, p['edge_embed_W'], b2(p['edge_embed_b']),
                   p['mask_embed'])

    tlw = p['tl_W'].transpose(2, 0, 1).reshape(_TD, _L * _H)
    tlb = p['tl_b'].reshape(1, _L * _H)
    tt_all = _tt_call(t.reshape(1, 1), tfreq,
                      p['time1_W'], b2(p['time1_b']),
                      p['time2_W'], b2(p['time2_b']), tlw, tlb)
    tt_all = tt_all.reshape(_L, _H)

    zeros_nh = jnp.zeros((_N, _H), _f32)
    sc_gather, sc_scatter = _sc_kernels()

    for i in range(_L):
        xd, xs = sc_gather(x, dst, src)
        ee, m = _edge_call(
            ee, xd, xs,
            p['C_W'][i], b2(p['C_b'][i]),
            p['A_W'][i], b2(p['A_b'][i]),
            p['B_W'][i], b2(p['B_b'][i]),
            p['V_W'][i], b2(p['V_b'][i]),
            b2(p['ln_e_g'][i]), b2(p['ln_e_b'][i]),
            tt_all[i].reshape(1, _H),
            b2(p['plo_g'][i]), b2(p['plo_b'][i]),
            p['plo_W'][i], b2(p['plo_b2'][i]),
        )
        agg = sc_scatter(m, src, zeros_nh)
        x = _node_call(x, agg, p['U_W'][i], b2(p['U_b'][i]),
                       b2(p['ln_h_g'][i]), b2(p['ln_h_b'][i]))

    st = _stats_call(ee)
    ow = jnp.zeros((8, _H), _f32).at[0:2].set(p['out_W'])
    ob = jnp.zeros((1, 8), _f32).at[0, 0:2].set(p['out_b'])
    out8 = _proj_call(ee, st, gmat, b2(p['gn_g']), b2(p['gn_b']), ow, ob)
    return (x, out8[:, 0:2])


# speedup vs baseline: 4.8711x; 1.2968x over previous
import functools

import jax
import jax.numpy as jnp
import numpy as np
from jax import lax
from jax.experimental import pallas as pl
from jax.experimental.pallas import tpu as pltpu
from jax.experimental.pallas import tpu_sc as plsc

_N = 10000
_E = 160000
_H = 128
_TD = _H // 2
_L = 12

_TE = 2000
_TN = 2000

_NC = 2
_NS = 16
_NW = _NC * _NS

_PW = _E // _NW
_PF = _PW // 128
_PR = _PW - _PF * 128
_ZR = 632
_ZL = _N - 15 * _ZR

_f32 = jnp.float32


_GC = 78
_GTAIL = 2 * _E // 128 - _NW * _GC
_HP = _H // 2


@functools.lru_cache(maxsize=None)
def _sc_kernels():
    mesh = plsc.VectorSubcoreMesh(core_axis_name="c", subcore_axis_name="s")

    @functools.partial(
        pl.kernel,
        out_type=jax.ShapeDtypeStruct((2 * _E, _H), _f32),
        mesh=mesh,
        scratch_types=[
            pltpu.VMEM((_GC * 128,), jnp.int32),
            pltpu.VMEM((_GTAIL * 128,), jnp.int32),
            pltpu.VMEM((6, 128, _H), _f32),
            pltpu.SemaphoreType.DMA,
            pltpu.SemaphoreType.DMA,
        ],
    )
    def sc_gather(x_hbm, idx_hbm, out_hbm, idx_v, idxt_v, bufs, sem_g, sem_w):
        wid = lax.axis_index("s") * _NC + lax.axis_index("c")
        base = wid * _GC * 128
        pltpu.sync_copy(idx_hbm.at[pl.ds(base, _GC * 128)], idx_v)

        def g_issue(c, bi):
            pltpu.async_copy(x_hbm.at[idx_v.at[pl.ds(c * 128, 128)]],
                             bufs.at[bi], sem_g)

        def g_wait(c, bi):
            pltpu.make_async_copy(x_hbm.at[idx_v.at[pl.ds(c * 128, 128)]],
                                  bufs.at[bi], sem_g).wait()

        def w_issue(c, bi):
            pltpu.async_copy(bufs.at[bi],
                             out_hbm.at[pl.ds(base + c * 128, 128)], sem_w)

        def w_wait(c, bi):
            pltpu.make_async_copy(bufs.at[bi],
                                  out_hbm.at[pl.ds(base + c * 128, 128)],
                                  sem_w).wait()

        @pl.loop(0, 13)
        def _(j):
            c0 = 6 * j

            @pl.when(j > 0)
            def _():
                for b in range(3):
                    w_wait(c0 - 6 + b, b)

            for b in range(3):
                g_issue(c0 + b, b)

            @pl.when(j > 0)
            def _():
                for b in range(3):
                    g_wait(c0 - 3 + b, 3 + b)
                for b in range(3):
                    w_issue(c0 - 3 + b, 3 + b)
                for b in range(3):
                    w_wait(c0 - 3 + b, 3 + b)

            for b in range(3):
                g_issue(c0 + 3 + b, 3 + b)

            for b in range(3):
                g_wait(c0 + b, b)
            for b in range(3):
                w_issue(c0 + b, b)

        for b in range(3):
            w_wait(72 + b, b)
        for b in range(3):
            g_wait(75 + b, 3 + b)
        for b in range(3):
            w_issue(75 + b, 3 + b)
        for b in range(3):
            w_wait(75 + b, 3 + b)

        @pl.when(wid == _NW - 1)
        def _():
            tbase = _NW * _GC * 128
            pltpu.sync_copy(idx_hbm.at[pl.ds(tbase, _GTAIL * 128)], idxt_v)
            for c in range(_GTAIL):
                pltpu.async_copy(
                    x_hbm.at[idxt_v.at[pl.ds(c * 128, 128)]],
                    bufs.at[c], sem_g).wait()
                pltpu.sync_copy(
                    bufs.at[c],
                    out_hbm.at[pl.ds(tbase + c * 128, 128)])

    @functools.partial(
        pl.kernel,
        out_type=jax.ShapeDtypeStruct((_NC, _N, _H), _f32),
        mesh=mesh,
        scratch_types=[
            pltpu.VMEM((128,), jnp.int32),
            pltpu.VMEM((128,), jnp.int32),
            pltpu.VMEM((128, _H), _f32),
            pltpu.VMEM((128, _H), _f32),
            pltpu.VMEM((_PR,), jnp.int32),
            pltpu.VMEM((_PR, _H), _f32),
            pltpu.VMEM_SHARED((_N, _H), _f32),
            pltpu.SemaphoreType.DMA,
            pltpu.SemaphoreType.DMA,
        ],
    )
    def sc_scatter(m_hbm, src_hbm, zero_hbm, out_hbm,
                   idx_a, idx_b, m_a, m_b, idx_r, rows_r, acc, sem_i, sem_m):
        cid = lax.axis_index("c")
        sid = lax.axis_index("s")
        z0 = sid * _ZR

        @pl.when(sid < 15)
        def _():
            pltpu.sync_copy(zero_hbm.at[pl.ds(z0, _ZR)], acc.at[pl.ds(z0, _ZR)])

        @pl.when(sid == 15)
        def _():
            pltpu.sync_copy(zero_hbm.at[pl.ds(z0, _ZL)], acc.at[pl.ds(z0, _ZL)])

        plsc.subcore_barrier()

        base = cid * (_E // _NC) + sid * _PW

        def l_issue(k, idx_v, m_v):
            off = base + k * 128
            pltpu.async_copy(src_hbm.at[pl.ds(off, 128)], idx_v, sem_i)
            pltpu.async_copy(m_hbm.at[pl.ds(off, 128)], m_v, sem_m)

        def l_wait(k, idx_v, m_v):
            off = base + k * 128
            pltpu.make_async_copy(src_hbm.at[pl.ds(off, 128)], idx_v,
                                  sem_i).wait()
            pltpu.make_async_copy(m_hbm.at[pl.ds(off, 128)], m_v,
                                  sem_m).wait()

        l_issue(0, idx_a, m_a)

        @pl.loop(0, (_PF - 1) // 2)
        def _(j):
            k = 2 * j
            l_issue(k + 1, idx_b, m_b)
            l_wait(k, idx_a, m_a)
            pltpu.sync_copy(m_a, acc.at[idx_a], add=True)
            l_issue(k + 2, idx_a, m_a)
            l_wait(k + 1, idx_b, m_b)
            pltpu.sync_copy(m_b, acc.at[idx_b], add=True)

        l_wait(_PF - 1, idx_a, m_a)
        pltpu.sync_copy(m_a, acc.at[idx_a], add=True)

        off = base + _PF * 128
        pltpu.sync_copy(src_hbm.at[pl.ds(off, _PR)], idx_r)
        pltpu.sync_copy(m_hbm.at[pl.ds(off, _PR)], rows_r)
        pltpu.sync_copy(rows_r, acc.at[idx_r], add=True)

        plsc.subcore_barrier()

        @pl.when(sid < 15)
        def _():
            pltpu.sync_copy(acc.at[pl.ds(z0, _ZR)],
                            out_hbm.at[cid, pl.ds(z0, _ZR)])

        @pl.when(sid == 15)
        def _():
            pltpu.sync_copy(acc.at[pl.ds(z0, _ZL)],
                            out_hbm.at[cid, pl.ds(z0, _ZL)])

    return sc_gather, sc_scatter


def _ln(x, g, b):
    mu = jnp.mean(x, axis=-1, keepdims=True)
    xc = x - mu
    var = jnp.mean(xc * xc, axis=-1, keepdims=True)
    return xc * lax.rsqrt(var + 1e-5) * g + b


def _mm(x, w):
    return lax.dot_general(x, w, (((1,), (1,)), ((), ())))


def _mmb(x, w):
    return lax.dot_general(x.astype(jnp.bfloat16), w.astype(jnp.bfloat16),
                           (((1,), (1,)), ((), ())),
                           preferred_element_type=jnp.float32)


def _edge_body(ee_ref, xd_ref, xs_ref, cw, cb, aw, ab, bw, bb, vw, vb,
               lneg, lneb, tt, plog, plob, plow, plob2, eeo_ref, m_ref):
    ee = ee_ref[...]
    xd = xd_ref[...]
    xs = xs_ref[...]
    ce = _mm(ee, cw[...]) + cb[...]
    ah = _mmb(xd, aw[...]) + ab[...]
    bh = _mmb(xs, bw[...]) + bb[...]
    vh = _mmb(xd, vw[...]) + vb[...]
    e_new = ah + bh + ce
    m_ref[...] = vh * jax.nn.sigmoid(e_new)
    t1 = jax.nn.relu(_ln(e_new, lneg[...], lneb[...]))
    t2 = t1 + tt[...]
    eo = _ln(t2, plog[...], plob[...])
    eo = eo * jax.nn.sigmoid(eo)
    eo = _mm(eo, plow[...]) + plob2[...]
    eeo_ref[...] = ee + eo


def _node_body(x_ref, agg_ref, uw, ub, lnhg, lnhb, xo_ref):
    x = x_ref[...]
    uh = _mm(x, uw[...]) + ub[...]
    s = uh + agg_ref[0] + agg_ref[1]
    xo_ref[...] = x + jax.nn.relu(_ln(s, lnhg[...], lnhb[...]))


def _x0_body(coords_ref, freq_ref, msk_ref, w, b, xo_ref):
    yc = coords_ref[:, 0:1]
    xc = coords_ref[:, 1:2]
    sel = msk_ref[0:1, :]
    par = msk_ref[1:2, :]
    c = yc * sel + xc * (1.0 - sel)
    arg = c * freq_ref[...]
    feat = jnp.where(par > 0.0, jnp.sin(arg), jnp.cos(arg))
    xo_ref[...] = _mm(feat, w[...]) + b[...]


def _ee0_body(e_ref, mask_ref, freq_ref, msk_ref, w, b, me, eeo_ref):
    arg = e_ref[...] * freq_ref[...]
    par = msk_ref[1:2, :]
    feat = jnp.where(par > 0.0, jnp.sin(arg), jnp.cos(arg))
    emb = _mm(feat, w[...]) + b[...]
    mk = mask_ref[...]
    me0 = me[0:1, :]
    me1 = me[1:2, :]
    eeo_ref[...] = emb + me0 + mk * (me1 - me0)


def _tt_body(t_ref, tfreq, w1, b1, w2, b2, tlw, tlb, out_ref):
    targ = t_ref[...] * tfreq[...]
    feat = jnp.concatenate([jnp.cos(targ), jnp.sin(targ)], axis=1)
    h1 = jax.nn.relu(_mm(feat, w1[...]) + b1[...])
    h2 = _mm(h1, w2[...]) + b2[...]
    rt = jax.nn.relu(h2)
    out_ref[...] = jnp.dot(rt, tlw[...]) + tlb[...]


def _stats_body(ee_ref, out_ref):
    @pl.when(pl.program_id(0) == 0)
    def _():
        out_ref[...] = jnp.zeros_like(out_ref)

    ee = ee_ref[...]
    out_ref[0:1, :] += jnp.sum(ee, axis=0, keepdims=True)
    out_ref[1:2, :] += jnp.sum(ee * ee, axis=0, keepdims=True)


def _proj_body(ee_ref, st_ref, gmat, gng, gnb, ow, ob, out_ref):
    mu = jnp.dot(st_ref[0:1, :], gmat[...])
    ms = jnp.dot(st_ref[1:2, :], gmat[...])
    var = ms - mu * mu
    inv = lax.rsqrt(var + 1e-5)
    yv = jax.nn.relu((ee_ref[...] - mu) * inv * gng[...] + gnb[...])
    out_ref[...] = _mm(yv, ow[...]) + ob[...]


def _full(i):
    return (0, 0)


def _rows(i):
    return (i, 0)


def _spec(shape, imap):
    return pl.BlockSpec(shape, imap)


_EDGE_GRID = _E // _TE
_NODE_GRID = _N // _TN

_edge_call = pl.pallas_call(
    _edge_body,
    grid=(_EDGE_GRID,),
    in_specs=[
        _spec((_TE, _H), _rows),
        _spec((_TE, _H), _rows),
        pl.BlockSpec((_TE, _H), lambda i: (i + _E // _TE, 0)),
        _spec((_H, _H), _full), _spec((1, _H), _full),
        _spec((_H, _H), _full), _spec((1, _H), _full),
        _spec((_H, _H), _full), _spec((1, _H), _full),
        _spec((_H, _H), _full), _spec((1, _H), _full),
        _spec((1, _H), _full), _spec((1, _H), _full),
        _spec((1, _H), _full),
        _spec((1, _H), _full), _spec((1, _H), _full),
        _spec((_H, _H), _full), _spec((1, _H), _full),
    ],
    out_specs=[_spec((_TE, _H), _rows), _spec((_TE, _H), _rows)],
    out_shape=[jax.ShapeDtypeStruct((_E, _H), _f32),
               jax.ShapeDtypeStruct((_E, _H), _f32)],
)

_node_call = pl.pallas_call(
    _node_body,
    grid=(_NODE_GRID,),
    in_specs=[
        _spec((_TN, _H), _rows),
        pl.BlockSpec((_NC, _TN, _H), lambda i: (0, i, 0)),
        _spec((_H, _H), _full), _spec((1, _H), _full),
        _spec((1, _H), _full), _spec((1, _H), _full),
    ],
    out_specs=_spec((_TN, _H), _rows),
    out_shape=jax.ShapeDtypeStruct((_N, _H), _f32),
)

_x0_call = pl.pallas_call(
    _x0_body,
    grid=(_NODE_GRID,),
    in_specs=[
        _spec((_TN, 2), _rows),
        _spec((1, _H), _full), _spec((2, _H), _full),
        _spec((_H, _H), _full), _spec((1, _H), _full),
    ],
    out_specs=_spec((_TN, _H), _rows),
    out_shape=jax.ShapeDtypeStruct((_N, _H), _f32),
)

_ee0_call = pl.pallas_call(
    _ee0_body,
    grid=(_EDGE_GRID,),
    in_specs=[
        _spec((_TE, 1), _rows), _spec((_TE, 1), _rows),
        _spec((1, _H), _full), _spec((2, _H), _full),
        _spec((_H, _H), _full), _spec((1, _H), _full),
        _spec((2, _H), _full),
    ],
    out_specs=_spec((_TE, _H), _rows),
    out_shape=jax.ShapeDtypeStruct((_E, _H), _f32),
)

_tt_call = pl.pallas_call(
    _tt_body,
    grid=(1,),
    in_specs=[
        _spec((1, 1), _full), _spec((1, _TD), _full),
        _spec((_TD, _H), _full), _spec((1, _TD), _full),
        _spec((_TD, _TD), _full), _spec((1, _TD), _full),
        _spec((_TD, _L * _H), _full), _spec((1, _L * _H), _full),
    ],
    out_specs=_spec((1, _L * _H), _full),
    out_shape=jax.ShapeDtypeStruct((1, _L * _H), _f32),
)

_stats_call = pl.pallas_call(
    _stats_body,
    grid=(_EDGE_GRID,),
    in_specs=[_spec((_TE, _H), _rows)],
    out_specs=_spec((8, _H), _full),
    out_shape=jax.ShapeDtypeStruct((8, _H), _f32),
)

_proj_call = pl.pallas_call(
    _proj_body,
    grid=(_EDGE_GRID,),
    in_specs=[
        _spec((_TE, _H), _rows), _spec((8, _H), _full),
        _spec((_H, _H), _full),
        _spec((1, _H), _full), _spec((1, _H), _full),
        _spec((8, _H), _full), _spec((1, 8), _full),
    ],
    out_specs=_spec((_TE, 8), _rows),
    out_shape=jax.ShapeDtypeStruct((_E, 8), _f32),
)


def _np_consts():
    npf = _H // 2
    dim64 = 10000.0 ** (2.0 * np.floor(np.arange(npf) / 2.0) / npf)
    freq_pos = np.tile(2.0 * np.pi / dim64, 2).astype(np.float32)[None, :]
    dim128 = 10000.0 ** (2.0 * np.floor(np.arange(_H) / 2.0) / _H)
    freq_e = (1.0 / dim128).astype(np.float32)[None, :]
    j = np.arange(_H)
    msk = np.stack([(j < npf).astype(np.float32),
                    (j % 2 == 0).astype(np.float32)]).astype(np.float32)
    tfreq = np.exp(-np.log(10000.0) * np.arange(_TD) / _TD).astype(np.float32)[None, :]
    gid = j // 4
    gmat = ((gid[:, None] == gid[None, :]).astype(np.float32) / (4.0 * _E))
    return freq_pos, freq_e, msk, tfreq, gmat


def kernel(nodes_feature, e, mask, t, edge_index, params):
    p = params
    freq_pos, freq_e, msk, tfreq, gmat = _np_consts()

    src = edge_index[0]
    dst = edge_index[1]
    idx_all = jnp.concatenate([dst, src], axis=0)

    def b2(v):
        return v.reshape(1, -1)

    x = _x0_call(nodes_feature, freq_pos, msk,
                 p['node_embed_W'], b2(p['node_embed_b']))
    ee = _ee0_call(e.reshape(_E, 1), mask.astype(_f32).reshape(_E, 1),
                   freq_e, msk, p['edge_embed_W'], b2(p['edge_embed_b']),
                   p['mask_embed'])

    tlw = p['tl_W'].transpose(2, 0, 1).reshape(_TD, _L * _H)
    tlb = p['tl_b'].reshape(1, _L * _H)
    tt_all = _tt_call(t.reshape(1, 1), tfreq,
                      p['time1_W'], b2(p['time1_b']),
                      p['time2_W'], b2(p['time2_b']), tlw, tlb)
    tt_all = tt_all.reshape(_L, _H)

    zeros_nh = jnp.zeros((_N, _H), _f32)
    sc_gather, sc_scatter = _sc_kernels()

    for i in range(_L):
        xgb = sc_gather(x, idx_all)
        ee, m = _edge_call(
            ee, xgb, xgb,
            p['C_W'][i], b2(p['C_b'][i]),
            p['A_W'][i], b2(p['A_b'][i]),
            p['B_W'][i], b2(p['B_b'][i]),
            p['V_W'][i], b2(p['V_b'][i]),
            b2(p['ln_e_g'][i]), b2(p['ln_e_b'][i]),
            tt_all[i].reshape(1, _H),
            b2(p['plo_g'][i]), b2(p['plo_b'][i]),
            p['plo_W'][i], b2(p['plo_b2'][i]),
        )
        agg = sc_scatter(m, src, zeros_nh)
        x = _node_call(x, agg, p['U_W'][i], b2(p['U_b'][i]),
                       b2(p['ln_h_g'][i]), b2(p['ln_h_b'][i]))

    st = _stats_call(ee)
    ow = jnp.zeros((8, _H), _f32).at[0:2].set(p['out_W'])
    ob = jnp.zeros((1, 8), _f32).at[0, 0:2].set(p['out_b'])
    out8 = _proj_call(ee, st, gmat, b2(p['gn_g']), b2(p['gn_b']), ow, ob)
    return (x, out8[:, 0:2])
